# Initial kernel scaffold; baseline (speedup 1.0000x reference)
import jax
import jax.numpy as jnp
from jax.experimental import pallas as pl


def kernel(premise, hypothesis, table, W, b):
    raise NotImplementedError("write your pallas kernel here")


# speedup vs baseline: 7.9831x; 7.9831x over previous
import functools

import jax
import jax.numpy as jnp
from jax import lax
from jax.experimental import pallas as pl
from jax.experimental.pallas import tpu as pltpu
from jax.experimental.pallas import tpu_sc as plsc

VOCAB = 100000
D = 128
BATCH = 4096
SEQ = 200

NC = 2
NS = 16
L = 16
NW = NC * NS
BPW = BATCH // NW
CHUNK = 100
NCHUNK = SEQ // CHUNK
NLANE = D // L


def _sc_kernel(pidx_hbm, hidx_hbm, table_hbm, wb_hbm, out_hbm,
               idx_v, rows_v, wb_v, logp_v, logh_v, outv_v, sem):
    wid = lax.axis_index("s") * NC + lax.axis_index("c")
    base = wid * BPW

    pltpu.sync_copy(wb_hbm, wb_v)

    def run_phase(side_hbm, w_off, log_ref):
        pltpu.sync_copy(side_hbm.at[wid], idx_v)
        wvecs = [wb_v[pl.ds(w_off + c * L, L)] for c in range(NLANE)]
        lane = lax.iota(jnp.int32, L)

        def sample_body(i, _):
            cp0 = pltpu.async_copy(
                table_hbm.at[idx_v.at[2 * i]], rows_v.at[pl.ds(0, CHUNK)], sem)
            cp1 = pltpu.async_copy(
                table_hbm.at[idx_v.at[2 * i + 1]],
                rows_v.at[pl.ds(CHUNK, CHUNK)], sem)
            cp0.wait()
            cp1.wait()

            neg_inf = jnp.full((L,), -jnp.inf, jnp.float32)

            def row_body(j, carry):
                return tuple(
                    jnp.maximum(carry[c], rows_v[j, pl.ds(c * L, L)])
                    for c in range(NLANE))

            maxes = lax.fori_loop(0, SEQ, row_body, (neg_inf,) * NLANE)
            acc = maxes[0] * wvecs[0]
            for c in range(1, NLANE):
                acc = acc + maxes[c] * wvecs[c]
            for sh in (1, 2, 4, 8):
                acc = acc + acc.at[lane ^ sh].get(mode="promise_in_bounds")
            log_ref[i, :] = acc
            return 0

        lax.fori_loop(0, BPW, sample_body, 0)

    run_phase(pidx_hbm, 0, logp_v)
    run_phase(hidx_hbm, D, logh_v)

    bvec = wb_v[pl.ds(2 * D, L)]
    lane = lax.iota(jnp.int32, L)
    for g in range(BPW // L):
        x = bvec
        for k in range(L):
            r = g * L + k
            x = jnp.where(lane == k,
                          x + logp_v[r, pl.ds(0, L)] + logh_v[r, pl.ds(0, L)],
                          x)
        outv_v[pl.ds(g * L, L)] = 1.0 / (1.0 + jnp.exp(-x))
    pltpu.sync_copy(outv_v, out_hbm.at[pl.ds(base, BPW)])


@jax.jit
def kernel(premise, hypothesis, table, W, b):
    pidx = premise.reshape(NW, BPW * NCHUNK, CHUNK)
    hidx = hypothesis.reshape(NW, BPW * NCHUNK, CHUNK)
    wb = jnp.concatenate([W.reshape(-1), jnp.broadcast_to(b, (L,))])

    mesh = plsc.VectorSubcoreMesh(core_axis_name="c", subcore_axis_name="s")
    f = pl.kernel(
        _sc_kernel,
        mesh=mesh,
        out_type=jax.ShapeDtypeStruct((BATCH,), jnp.float32),
        scratch_types=[
            pltpu.VMEM((BPW * NCHUNK, CHUNK), jnp.int32),
            pltpu.VMEM((SEQ, D), jnp.float32),
            pltpu.VMEM((2 * D + L,), jnp.float32),
            pltpu.VMEM((BPW, L), jnp.float32),
            pltpu.VMEM((BPW, L), jnp.float32),
            pltpu.VMEM((BPW,), jnp.float32),
            pltpu.SemaphoreType.DMA,
        ],
    )
    return f(pidx, hidx, table, wb)


# speedup vs baseline: 14.5088x; 1.8175x over previous
import functools

import jax
import jax.numpy as jnp
from jax import lax
from jax.experimental import pallas as pl
from jax.experimental.pallas import tpu as pltpu
from jax.experimental.pallas import tpu_sc as plsc

VOCAB = 100000
D = 128
BATCH = 4096
SEQ = 200

NC = 2
NS = 16
L = 16
NW = NC * NS
BPW = BATCH // NW
CHUNK = 100
NCHUNK = SEQ // CHUNK
NLANE = D // L


def _sc_kernel(pidx_hbm, hidx_hbm, table_hbm, wb_hbm, out_hbm,
               idx_v, rows_v, wb_v, logp_v, logh_v, outv_v, sem0, sem1):
    wid = lax.axis_index("s") * NC + lax.axis_index("c")
    base = wid * BPW
    sems = (sem0, sem1)
    lane = lax.iota(jnp.int32, L)
    neg_inf = jnp.full((L,), -jnp.inf, jnp.float32)

    pltpu.sync_copy(wb_hbm, wb_v)

    def copies(i, b):
        return (
            pltpu.make_async_copy(table_hbm.at[idx_v.at[2 * i]],
                                  rows_v.at[b, pl.ds(0, CHUNK)], sems[b]),
            pltpu.make_async_copy(table_hbm.at[idx_v.at[2 * i + 1]],
                                  rows_v.at[b, pl.ds(CHUNK, CHUNK)], sems[b]),
        )

    def issue(i, b):
        for cp in copies(i, b):
            cp.start()

    def wait(i, b):
        for cp in copies(i, b):
            cp.wait()

    def run_phase(side_hbm, w_off, log_ref):
        pltpu.sync_copy(side_hbm.at[wid], idx_v)
        wvecs = [wb_v[pl.ds(w_off + c * L, L)] for c in range(NLANE)]

        issue(0, 0)

        def pair_body(i2, _):
            for b in (0, 1):
                i = 2 * i2 + b

                @pl.when(i + 1 < BPW)
                def _():
                    issue(i + 1, 1 - b)

                wait(i, b)

                def row_body(j, carry):
                    return tuple(
                        jnp.maximum(carry[c], rows_v[b, j, pl.ds(c * L, L)])
                        for c in range(NLANE))

                maxes = lax.fori_loop(0, SEQ, row_body, (neg_inf,) * NLANE)
                acc = maxes[0] * wvecs[0]
                for c in range(1, NLANE):
                    acc = acc + maxes[c] * wvecs[c]
                for sh in (1, 2, 4, 8):
                    acc = acc + acc.at[lane ^ sh].get(
                        mode="promise_in_bounds")
                log_ref[i, :] = acc
            return 0

        lax.fori_loop(0, BPW // 2, pair_body, 0)

    run_phase(pidx_hbm, 0, logp_v)
    run_phase(hidx_hbm, D, logh_v)

    bvec = wb_v[pl.ds(2 * D, L)]
    lane = lax.iota(jnp.int32, L)
    for g in range(BPW // L):
        x = bvec
        for k in range(L):
            r = g * L + k
            x = jnp.where(lane == k,
                          x + logp_v[r, pl.ds(0, L)] + logh_v[r, pl.ds(0, L)],
                          x)
        outv_v[pl.ds(g * L, L)] = 1.0 / (1.0 + jnp.exp(-x))
    pltpu.sync_copy(outv_v, out_hbm.at[pl.ds(base, BPW)])


@jax.jit
def kernel(premise, hypothesis, table, W, b):
    pidx = premise.reshape(NW, BPW * NCHUNK, CHUNK)
    hidx = hypothesis.reshape(NW, BPW * NCHUNK, CHUNK)
    wb = jnp.concatenate([W.reshape(-1), jnp.broadcast_to(b, (L,))])

    mesh = plsc.VectorSubcoreMesh(core_axis_name="c", subcore_axis_name="s")
    f = pl.kernel(
        _sc_kernel,
        mesh=mesh,
        out_type=jax.ShapeDtypeStruct((BATCH,), jnp.float32),
        scratch_types=[
            pltpu.VMEM((BPW * NCHUNK, CHUNK), jnp.int32),
            pltpu.VMEM((2, SEQ, D), jnp.float32),
            pltpu.VMEM((2 * D + L,), jnp.float32),
            pltpu.VMEM((BPW, L), jnp.float32),
            pltpu.VMEM((BPW, L), jnp.float32),
            pltpu.VMEM((BPW,), jnp.float32),
            pltpu.SemaphoreType.DMA,
            pltpu.SemaphoreType.DMA,
        ],
    )
    return f(pidx, hidx, table, wb)
